# Initial kernel scaffold; baseline (speedup 1.0000x reference)
import jax
import jax.numpy as jnp
from jax.experimental import pallas as pl


def kernel(users, items, edge_index, edge_weight, user_emb, item_emb):
    raise NotImplementedError("write your pallas kernel here")


# speedup vs baseline: 1.9417x; 1.9417x over previous
import functools

import jax
import jax.numpy as jnp
from jax import lax
from jax.experimental import pallas as pl
from jax.experimental.pallas import tpu as pltpu
from jax.experimental.pallas import tpu_sc as plsc

N_USERS = 10000
N_ITEMS = 40000
N_NODES = N_USERS + N_ITEMS
D = 64
N_EDGES = 800000
HALF = N_NODES // 2
ACC_ROWS = HALF + 8
PAD_ROW = HALF
EC = 128
N_CHUNKS = N_EDGES // EC
WCHUNK = 40
N_WCHUNKS = HALF // WCHUNK

_mesh = plsc.VectorSubcoreMesh(core_axis_name="c", subcore_axis_name="s")
_cparams = pltpu.CompilerParams(use_tc_tiling_on_sc=False,
                                needs_layout_passes=False)


@functools.partial(
    pl.kernel,
    mesh=_mesh,
    out_type=[
        jax.ShapeDtypeStruct((N_NODES, D), jnp.float32),
        jax.ShapeDtypeStruct((N_NODES, D), jnp.float32),
    ],
    scratch_types=[
        pltpu.VMEM_SHARED((ACC_ROWS, D), jnp.float32),
        pltpu.VMEM((EC,), jnp.int32),
        pltpu.VMEM((EC,), jnp.int32),
        pltpu.VMEM((EC,), jnp.float32),
        pltpu.VMEM((EC,), jnp.float32),
        pltpu.VMEM((EC,), jnp.int32),
        pltpu.VMEM((EC, D), jnp.float32),
        pltpu.VMEM((WCHUNK, D), jnp.float32),
        pltpu.VMEM((WCHUNK, D), jnp.float32),
        pltpu.SemaphoreType.DMA,
    ],
    compiler_params=_cparams,
)
def _layer(x_hbm, src_hbm, dst_hbm, w_hbm, sum_hbm,
           xo_hbm, so_hbm,
           acc, srcc, dstc, wc, weff, ldst, rows, rb, sb, sem):
    cid = lax.axis_index("c")
    sid = lax.axis_index("s")
    base_node = cid * HALF

    def zset(i, _):
        for dd in range(4):
            rb[i, pl.ds(dd * 16, 16)] = jnp.zeros((16,), jnp.float32)
        return 0
    lax.fori_loop(0, WCHUNK, zset, 0)

    def zb(t, _):
        c = sid + t * 16
        pltpu.sync_copy(rb, acc.at[pl.ds(c * WCHUNK, WCHUNK)])
        return 0
    n_z = 39 + jnp.where(sid < 1, 1, 0)
    lax.fori_loop(0, n_z, zb, 0)
    @pl.when(sid == 0)
    def _():
        pltpu.sync_copy(rb.at[pl.ds(0, 8)], acc.at[pl.ds(HALF, 8)])

    plsc.subcore_barrier()

    def chunk_body(t, _):
        j = sid + t * 16
        off = j * EC
        pltpu.sync_copy(src_hbm.at[pl.ds(off, EC)], srcc)
        pltpu.sync_copy(dst_hbm.at[pl.ds(off, EC)], dstc)
        pltpu.sync_copy(w_hbm.at[pl.ds(off, EC)], wc)

        def cmp16(k, _):
            dv = dstc[pl.ds(k * 16, 16)] - base_node
            ok = (dv >= 0) & (dv < HALF)
            wv = wc[pl.ds(k * 16, 16)]
            weff[pl.ds(k * 16, 16)] = jnp.where(ok, wv, 0.0)
            ldst[pl.ds(k * 16, 16)] = jnp.where(ok, dv, PAD_ROW)
            return 0
        lax.fori_loop(0, EC // 16, cmp16, 0, unroll=2)

        pltpu.async_copy(x_hbm.at[srcc], rows, sem).wait()

        def mul_grp(k, _):
            wv = weff[pl.ds(k * 16, 16)]
            for l in range(16):
                i = k * 16 + l
                ws = wv[l]
                for dd in range(4):
                    rows[i, pl.ds(dd * 16, 16)] = (
                        rows[i, pl.ds(dd * 16, 16)] * ws)
            return 0
        lax.fori_loop(0, EC // 16, mul_grp, 0)

        pltpu.sync_copy(rows, acc.at[ldst], add=True)
        return 0

    n_c = 390 + jnp.where(sid < N_CHUNKS - 390 * 16, 1, 0)
    lax.fori_loop(0, n_c, chunk_body, 0)

    plsc.subcore_barrier()

    def wb(t, _):
        c = sid + t * 16
        r0 = c * WCHUNK
        pltpu.sync_copy(acc.at[pl.ds(r0, WCHUNK)], rb)
        pltpu.sync_copy(sum_hbm.at[pl.ds(base_node + r0, WCHUNK)], sb)

        def addrow(i, _):
            for dd in range(4):
                s = pl.ds(dd * 16, 16)
                sb[i, s] = sb[i, s] + rb[i, s]
            return 0
        lax.fori_loop(0, WCHUNK, addrow, 0, unroll=4)

        pltpu.sync_copy(rb, xo_hbm.at[pl.ds(base_node + r0, WCHUNK)])
        pltpu.sync_copy(sb, so_hbm.at[pl.ds(base_node + r0, WCHUNK)])
        return 0
    n_w = 39 + jnp.where(sid < 1, 1, 0)
    lax.fori_loop(0, n_w, wb, 0)


UPT = 32
IPAD = 112


@functools.partial(
    pl.kernel,
    mesh=_mesh,
    out_type=jax.ShapeDtypeStruct((1024, IPAD), jnp.float32),
    scratch_types=[
        pltpu.VMEM((UPT,), jnp.int32),
        pltpu.VMEM((UPT, D), jnp.float32),
        pltpu.VMEM((224,), jnp.int32),
        pltpu.VMEM((2, IPAD), jnp.int32),
        pltpu.VMEM((IPAD, D), jnp.float32),
        pltpu.VMEM((IPAD,), jnp.float32),
        pltpu.SemaphoreType.DMA,
    ],
    compiler_params=_cparams,
)
def _rate(sum_hbm, users_hbm, itemsf_hbm,
          out_hbm,
          uidx, urows, pairb, iidx, irows, rbuf, sem):
    cid = lax.axis_index("c")
    sid = lax.axis_index("s")
    wid = sid * 2 + cid
    ubase = wid * UPT

    pltpu.sync_copy(users_hbm.at[pl.ds(ubase, UPT)], uidx)
    pltpu.async_copy(sum_hbm.at[uidx], urows, sem).wait()

    lane = lax.iota(jnp.int32, 16)

    def pair_body(p, _):
        u_glob = ubase + 2 * p
        pltpu.sync_copy(itemsf_hbm.at[pl.ds(u_glob * 100, 200)],
                        pairb.at[pl.ds(0, 200)])

        for u in range(2):
            def fill(k, _):
                v = pairb[pl.ds(u * 100 + k * 16, 16)]
                v = jnp.clip(v + N_USERS, 0, N_NODES - 1)
                iidx[u, pl.ds(k * 16, 16)] = v
                return 0
            lax.fori_loop(0, IPAD // 16, fill, 0)

        for u in range(2):
            pltpu.async_copy(sum_hbm.at[iidx.at[u]], irows, sem).wait()
            ur = 2 * p + u

            def dot_d(dg, accs):
                uv = urows[ur, pl.ds(dg * 16, 16)]
                for dl in range(16):
                    d = dg * 16 + dl
                    ud = uv[dl]
                    new = []
                    for g in range(IPAD // 16):
                        col = plsc.load_gather(
                            irows,
                            [g * 16 + lane, jnp.full((16,), d, jnp.int32)])
                        new.append(accs[g] + ud * col)
                    accs = tuple(new)
                return accs

            accs = lax.fori_loop(
                0, D // 16, dot_d,
                tuple(jnp.zeros((16,), jnp.float32) for _ in range(IPAD // 16)))

            for g in range(IPAD // 16):
                r = accs[g] * (1.0 / 16.0)
                s = 1.0 / (1.0 + jnp.exp(-r))
                rbuf[pl.ds(g * 16, 16)] = s
            pltpu.sync_copy(rbuf, out_hbm.at[u_glob + u])
        return 0

    lax.fori_loop(0, UPT // 2, pair_body, 0)


def kernel(users, items, edge_index, edge_weight, user_emb, item_emb):
    all_emb = jnp.concatenate([user_emb, item_emb], axis=0)
    src = edge_index[0]
    dst = edge_index[1]
    x = all_emb
    s = all_emb
    for _ in range(3):
        x, s = _layer(x, src, dst, edge_weight, s)
    out = _rate(s, users, items.reshape(-1))
    return out[:, :100]


# speedup vs baseline: 2.7268x; 1.4043x over previous
import functools

import jax
import jax.numpy as jnp
from jax import lax
from jax.experimental import pallas as pl
from jax.experimental.pallas import tpu as pltpu
from jax.experimental.pallas import tpu_sc as plsc

N_USERS = 10000
N_ITEMS = 40000
N_NODES = N_USERS + N_ITEMS
D = 64
N_EDGES = 800000
HALF = N_NODES // 2
ACC_ROWS = HALF + 8
PAD_ROW = HALF
EC = 128
NQ = 392
N_CHUNKS = NQ * 16
E_PAD = N_CHUNKS * EC
WCHUNK = 40

_mesh = plsc.VectorSubcoreMesh(core_axis_name="c", subcore_axis_name="s")
_cparams = pltpu.CompilerParams(use_tc_tiling_on_sc=False,
                                needs_layout_passes=False)


@functools.partial(
    pl.kernel,
    mesh=_mesh,
    out_type=[
        jax.ShapeDtypeStruct((N_NODES, D), jnp.float32),
        jax.ShapeDtypeStruct((N_NODES, D), jnp.float32),
    ],
    scratch_types=[
        pltpu.VMEM_SHARED((ACC_ROWS, D), jnp.float32),
        [pltpu.VMEM((3, EC), jnp.int32)] * 2,
        [pltpu.VMEM((EC,), jnp.float32)] * 2,
        [pltpu.VMEM((EC,), jnp.int32)] * 2,
        [pltpu.VMEM((EC, D), jnp.float32)] * 2,
        pltpu.VMEM((WCHUNK, D), jnp.float32),
        pltpu.VMEM((WCHUNK, D), jnp.float32),
        [pltpu.SemaphoreType.DMA] * 2,
        [pltpu.SemaphoreType.DMA] * 2,
        [pltpu.SemaphoreType.DMA] * 2,
    ],
    compiler_params=_cparams,
)
def _layer(x_hbm, pk_hbm, sum_hbm,
           xo_hbm, so_hbm,
           acc, idxb, weff, ldst, rows, rb, sb, sem_i, sem_g, sem_s):
    cid = lax.axis_index("c")
    sid = lax.axis_index("s")
    base_node = cid * HALF

    def zset(i, _):
        for dd in range(4):
            rb[i, pl.ds(dd * 16, 16)] = jnp.zeros((16,), jnp.float32)
        return 0
    lax.fori_loop(0, WCHUNK, zset, 0)

    def zb(t, _):
        c = sid + t * 16
        pltpu.sync_copy(rb, acc.at[pl.ds(c * WCHUNK, WCHUNK)])
        return 0
    n_z = 39 + jnp.where(sid < 1, 1, 0)
    lax.fori_loop(0, n_z, zb, 0)
    @pl.when(sid == 0)
    def _():
        pltpu.sync_copy(rb.at[pl.ds(0, 8)], acc.at[pl.ds(HALF, 8)])

    plsc.subcore_barrier()

    for b in range(2):
        pltpu.async_copy(pk_hbm.at[sid + b * 16], idxb[b], sem_i[b])

    def super_body(g, _):
        for b in range(2):
            q = 2 * g + b
            j = sid + q * 16
            pltpu.make_async_copy(pk_hbm.at[j], idxb[b], sem_i[b]).wait()
            @pl.when(g > 0)
            def _():
                pltpu.make_async_copy(rows[b], acc.at[ldst[b]],
                                      sem_s[b]).wait()
            gd = pltpu.async_copy(x_hbm.at[idxb[b].at[0]], rows[b], sem_g[b])

            def cmp16(k, _):
                s16 = pl.ds(k * 16, 16)
                dv = idxb[b][1, s16] - base_node
                ok = (dv >= 0) & (dv < HALF)
                wv = plsc.bitcast(idxb[b][2, s16], jnp.float32)
                weff[b][s16] = jnp.where(ok, wv, 0.0)
                ldst[b][s16] = jnp.where(ok, dv, PAD_ROW)
                return 0
            lax.fori_loop(0, EC // 16, cmp16, 0, unroll=4)

            gd.wait()
            @pl.when(q + 2 < NQ)
            def _():
                pltpu.async_copy(pk_hbm.at[j + 32], idxb[b], sem_i[b])

            def mul_grp(k, _):
                wv = weff[b][pl.ds(k * 16, 16)]
                for l in range(16):
                    i = k * 16 + l
                    ws = wv[l]
                    for dd in range(4):
                        s16 = pl.ds(dd * 16, 16)
                        rows[b][i, s16] = rows[b][i, s16] * ws
                return 0
            lax.fori_loop(0, EC // 16, mul_grp, 0)

            pltpu.async_copy(rows[b], acc.at[ldst[b]], sem_s[b], add=True)
        return 0

    lax.fori_loop(0, NQ // 2, super_body, 0)
    for b in range(2):
        pltpu.make_async_copy(rows[b], acc.at[ldst[b]], sem_s[b]).wait()

    plsc.subcore_barrier()

    def wb(t, _):
        c = sid + t * 16
        r0 = c * WCHUNK
        pltpu.sync_copy(acc.at[pl.ds(r0, WCHUNK)], rb)
        pltpu.sync_copy(sum_hbm.at[pl.ds(base_node + r0, WCHUNK)], sb)

        def addrow(i, _):
            for dd in range(4):
                s = pl.ds(dd * 16, 16)
                sb[i, s] = sb[i, s] + rb[i, s]
            return 0
        lax.fori_loop(0, WCHUNK, addrow, 0, unroll=4)

        pltpu.sync_copy(rb, xo_hbm.at[pl.ds(base_node + r0, WCHUNK)])
        pltpu.sync_copy(sb, so_hbm.at[pl.ds(base_node + r0, WCHUNK)])
        return 0
    n_w = 39 + jnp.where(sid < 1, 1, 0)
    lax.fori_loop(0, n_w, wb, 0)


UPT = 32
IPAD = 112


@functools.partial(
    pl.kernel,
    mesh=_mesh,
    out_type=jax.ShapeDtypeStruct((1024, IPAD), jnp.float32),
    scratch_types=[
        pltpu.VMEM((UPT,), jnp.int32),
        pltpu.VMEM((UPT, D), jnp.float32),
        pltpu.VMEM((224,), jnp.int32),
        pltpu.VMEM((2, IPAD), jnp.int32),
        pltpu.VMEM((IPAD, D), jnp.float32),
        pltpu.VMEM((IPAD,), jnp.float32),
        pltpu.SemaphoreType.DMA,
    ],
    compiler_params=_cparams,
)
def _rate(sum_hbm, users_hbm, itemsf_hbm,
          out_hbm,
          uidx, urows, pairb, iidx, irows, rbuf, sem):
    cid = lax.axis_index("c")
    sid = lax.axis_index("s")
    wid = sid * 2 + cid
    ubase = wid * UPT

    pltpu.sync_copy(users_hbm.at[pl.ds(ubase, UPT)], uidx)
    pltpu.async_copy(sum_hbm.at[uidx], urows, sem).wait()

    lane = lax.iota(jnp.int32, 16)

    def pair_body(p, _):
        u_glob = ubase + 2 * p
        pltpu.sync_copy(itemsf_hbm.at[pl.ds(u_glob * 100, 200)],
                        pairb.at[pl.ds(0, 200)])

        for u in range(2):
            def fill(k, _):
                v = pairb[pl.ds(u * 100 + k * 16, 16)]
                v = jnp.clip(v + N_USERS, 0, N_NODES - 1)
                iidx[u, pl.ds(k * 16, 16)] = v
                return 0
            lax.fori_loop(0, IPAD // 16, fill, 0)

        for u in range(2):
            pltpu.async_copy(sum_hbm.at[iidx.at[u]], irows, sem).wait()
            ur = 2 * p + u

            def dot_d(dg, accs):
                uv = urows[ur, pl.ds(dg * 16, 16)]
                for dl in range(16):
                    d = dg * 16 + dl
                    ud = uv[dl]
                    new = []
                    for g in range(IPAD // 16):
                        col = plsc.load_gather(
                            irows,
                            [g * 16 + lane, jnp.full((16,), d, jnp.int32)])
                        new.append(accs[g] + ud * col)
                    accs = tuple(new)
                return accs

            accs = lax.fori_loop(
                0, D // 16, dot_d,
                tuple(jnp.zeros((16,), jnp.float32) for _ in range(IPAD // 16)))

            for g in range(IPAD // 16):
                r = accs[g] * (1.0 / 16.0)
                s = 1.0 / (1.0 + jnp.exp(-r))
                rbuf[pl.ds(g * 16, 16)] = s
            pltpu.sync_copy(rbuf, out_hbm.at[u_glob + u])
        return 0

    lax.fori_loop(0, UPT // 2, pair_body, 0)


def kernel(users, items, edge_index, edge_weight, user_emb, item_emb):
    all_emb = jnp.concatenate([user_emb, item_emb], axis=0)
    src = edge_index[0]
    dst = edge_index[1]
    wbits = lax.bitcast_convert_type(edge_weight, jnp.int32)
    pad = E_PAD - N_EDGES
    zpad = jnp.zeros((pad,), jnp.int32)
    packed = jnp.stack([
        jnp.concatenate([src, zpad]),
        jnp.concatenate([dst, zpad]),
        jnp.concatenate([wbits, zpad]),
    ], axis=0).reshape(3, N_CHUNKS, EC).transpose(1, 0, 2)

    x = all_emb
    s = all_emb
    for _ in range(3):
        x, s = _layer(x, packed, s)
    out = _rate(s, users, items.reshape(-1))
    return out[:, :100]


# speedup vs baseline: 4.9835x; 1.8276x over previous
import functools

import jax
import jax.numpy as jnp
from jax import lax
from jax.experimental import pallas as pl
from jax.experimental.pallas import tpu as pltpu
from jax.experimental.pallas import tpu_sc as plsc

N_USERS = 10000
N_ITEMS = 40000
N_NODES = N_USERS + N_ITEMS
D = 64
N_EDGES = 800000
HALF = N_NODES // 2
ACC_ROWS = HALF + 8
PAD_ROW = HALF
EC = 128
NQ = 392
N_CHUNKS = NQ * 16
E_PAD = N_CHUNKS * EC
WCHUNK = 40

_mesh = plsc.VectorSubcoreMesh(core_axis_name="c", subcore_axis_name="s")
_cparams = pltpu.CompilerParams(use_tc_tiling_on_sc=False,
                                needs_layout_passes=False)


@functools.partial(
    pl.kernel,
    mesh=_mesh,
    out_type=[
        jax.ShapeDtypeStruct((N_NODES, D), jnp.float32),
        jax.ShapeDtypeStruct((N_NODES, D), jnp.float32),
    ],
    scratch_types=[
        pltpu.VMEM_SHARED((ACC_ROWS, D), jnp.float32),
        [pltpu.VMEM((3, EC), jnp.int32)] * 2,
        [pltpu.VMEM((EC,), jnp.float32)] * 2,
        [pltpu.VMEM((EC,), jnp.int32)] * 2,
        [pltpu.VMEM((EC, D), jnp.float32)] * 2,
        pltpu.VMEM((WCHUNK, D), jnp.float32),
        pltpu.VMEM((WCHUNK, D), jnp.float32),
        [pltpu.SemaphoreType.DMA] * 2,
        [pltpu.SemaphoreType.DMA] * 2,
        [pltpu.SemaphoreType.DMA] * 2,
    ],
    compiler_params=_cparams,
)
def _layer(x_hbm, pk_hbm, sum_hbm,
           xo_hbm, so_hbm,
           acc, idxb, weff, ldst, rows, rb, sb, sem_i, sem_g, sem_s):
    cid = lax.axis_index("c")
    sid = lax.axis_index("s")
    base_node = cid * HALF

    def zset(i, _):
        for dd in range(4):
            rb[i, pl.ds(dd * 16, 16)] = jnp.zeros((16,), jnp.float32)
        return 0
    lax.fori_loop(0, WCHUNK, zset, 0)

    def zb(t, _):
        c = sid + t * 16
        pltpu.sync_copy(rb, acc.at[pl.ds(c * WCHUNK, WCHUNK)])
        return 0
    n_z = 39 + jnp.where(sid < 1, 1, 0)
    lax.fori_loop(0, n_z, zb, 0)
    @pl.when(sid == 0)
    def _():
        pltpu.sync_copy(rb.at[pl.ds(0, 8)], acc.at[pl.ds(HALF, 8)])

    plsc.subcore_barrier()

    for b in range(2):
        pltpu.async_copy(pk_hbm.at[sid + b * 16], idxb[b], sem_i[b])

    def super_body(g, _):
        for b in range(2):
            q = 2 * g + b
            j = sid + q * 16
            pltpu.make_async_copy(pk_hbm.at[j], idxb[b], sem_i[b]).wait()
            @pl.when(g > 0)
            def _():
                pltpu.make_async_copy(rows[b], acc.at[ldst[b]],
                                      sem_s[b]).wait()
            pltpu.async_copy(x_hbm.at[idxb[b].at[0]], rows[b], sem_g[b])

            @plsc.parallel_loop(0, EC // 16, unroll=4)
            def _(k):
                s16 = pl.ds(k * 16, 16)
                dv = idxb[b][1, s16] - base_node
                ok = (dv >= 0) & (dv < HALF)
                wv = plsc.bitcast(idxb[b][2, s16], jnp.float32)
                weff[b][s16] = jnp.where(ok, wv, 0.0)
                ldst[b][s16] = jnp.where(ok, dv, PAD_ROW)

        for b in range(2):
            q = 2 * g + b
            j = sid + q * 16
            pltpu.make_async_copy(x_hbm.at[idxb[b].at[0]], rows[b],
                                  sem_g[b]).wait()
            @pl.when(q + 2 < NQ)
            def _():
                pltpu.async_copy(pk_hbm.at[j + 32], idxb[b], sem_i[b])

            @plsc.parallel_loop(0, EC // 16, unroll=4)
            def _(k):
                wv = weff[b][pl.ds(k * 16, 16)]
                for l in range(16):
                    i = k * 16 + l
                    ws = wv[l]
                    for dd in range(4):
                        s16 = pl.ds(dd * 16, 16)
                        rows[b][i, s16] = rows[b][i, s16] * ws

            pltpu.async_copy(rows[b], acc.at[ldst[b]], sem_s[b], add=True)
        return 0

    lax.fori_loop(0, NQ // 2, super_body, 0)
    for b in range(2):
        pltpu.make_async_copy(rows[b], acc.at[ldst[b]], sem_s[b]).wait()

    plsc.subcore_barrier()

    def wb(t, _):
        c = sid + t * 16
        r0 = c * WCHUNK
        pltpu.sync_copy(acc.at[pl.ds(r0, WCHUNK)], rb)
        pltpu.sync_copy(sum_hbm.at[pl.ds(base_node + r0, WCHUNK)], sb)

        @plsc.parallel_loop(0, WCHUNK, unroll=4)
        def _(i):
            for dd in range(4):
                s = pl.ds(dd * 16, 16)
                sb[i, s] = sb[i, s] + rb[i, s]

        pltpu.sync_copy(rb, xo_hbm.at[pl.ds(base_node + r0, WCHUNK)])
        pltpu.sync_copy(sb, so_hbm.at[pl.ds(base_node + r0, WCHUNK)])
        return 0
    n_w = 39 + jnp.where(sid < 1, 1, 0)
    lax.fori_loop(0, n_w, wb, 0)


UPT = 32
IPAD = 112


@functools.partial(
    pl.kernel,
    mesh=_mesh,
    out_type=jax.ShapeDtypeStruct((1024, IPAD), jnp.float32),
    scratch_types=[
        pltpu.VMEM((UPT,), jnp.int32),
        pltpu.VMEM((UPT, D), jnp.float32),
        pltpu.VMEM((224,), jnp.int32),
        pltpu.VMEM((2, IPAD), jnp.int32),
        pltpu.VMEM((IPAD, D), jnp.float32),
        pltpu.VMEM((IPAD,), jnp.float32),
        pltpu.SemaphoreType.DMA,
    ],
    compiler_params=_cparams,
)
def _rate(sum_hbm, users_hbm, itemsf_hbm,
          out_hbm,
          uidx, urows, pairb, iidx, irows, rbuf, sem):
    cid = lax.axis_index("c")
    sid = lax.axis_index("s")
    wid = sid * 2 + cid
    ubase = wid * UPT

    pltpu.sync_copy(users_hbm.at[pl.ds(ubase, UPT)], uidx)
    pltpu.async_copy(sum_hbm.at[uidx], urows, sem).wait()

    lane = lax.iota(jnp.int32, 16)

    def pair_body(p, _):
        u_glob = ubase + 2 * p
        pltpu.sync_copy(itemsf_hbm.at[pl.ds(u_glob * 100, 200)],
                        pairb.at[pl.ds(0, 200)])

        for u in range(2):
            def fill(k, _):
                v = pairb[pl.ds(u * 100 + k * 16, 16)]
                v = jnp.clip(v + N_USERS, 0, N_NODES - 1)
                iidx[u, pl.ds(k * 16, 16)] = v
                return 0
            lax.fori_loop(0, IPAD // 16, fill, 0)

        for u in range(2):
            pltpu.async_copy(sum_hbm.at[iidx.at[u]], irows, sem).wait()
            ur = 2 * p + u

            def dot_d(dg, accs):
                uv = urows[ur, pl.ds(dg * 16, 16)]
                for dl in range(16):
                    d = dg * 16 + dl
                    ud = uv[dl]
                    new = []
                    for g in range(IPAD // 16):
                        col = plsc.load_gather(
                            irows,
                            [g * 16 + lane, jnp.full((16,), d, jnp.int32)])
                        new.append(accs[g] + ud * col)
                    accs = tuple(new)
                return accs

            accs = lax.fori_loop(
                0, D // 16, dot_d,
                tuple(jnp.zeros((16,), jnp.float32) for _ in range(IPAD // 16)))

            for g in range(IPAD // 16):
                r = accs[g] * (1.0 / 16.0)
                s = 1.0 / (1.0 + jnp.exp(-r))
                rbuf[pl.ds(g * 16, 16)] = s
            pltpu.sync_copy(rbuf, out_hbm.at[u_glob + u])
        return 0

    lax.fori_loop(0, UPT // 2, pair_body, 0)


def kernel(users, items, edge_index, edge_weight, user_emb, item_emb):
    all_emb = jnp.concatenate([user_emb, item_emb], axis=0)
    src = edge_index[0]
    dst = edge_index[1]
    wbits = lax.bitcast_convert_type(edge_weight, jnp.int32)
    pad = E_PAD - N_EDGES
    zpad = jnp.zeros((pad,), jnp.int32)
    packed = jnp.stack([
        jnp.concatenate([src, zpad]),
        jnp.concatenate([dst, zpad]),
        jnp.concatenate([wbits, zpad]),
    ], axis=0).reshape(3, N_CHUNKS, EC).transpose(1, 0, 2)

    x = all_emb
    s = all_emb
    for _ in range(3):
        x, s = _layer(x, packed, s)
    out = _rate(s, users, items.reshape(-1))
    return out[:, :100]


# speedup vs baseline: 7.9610x; 1.5975x over previous
import functools

import jax
import jax.numpy as jnp
from jax import lax
from jax.experimental import pallas as pl
from jax.experimental.pallas import tpu as pltpu
from jax.experimental.pallas import tpu_sc as plsc

N_USERS = 10000
N_ITEMS = 40000
N_NODES = N_USERS + N_ITEMS
D = 64
N_EDGES = 800000
HALF = N_NODES // 2
ACC_ROWS = HALF + 8
PAD_ROW = HALF
EC = 128
NQ = 392
N_CHUNKS = NQ * 16
E_PAD = N_CHUNKS * EC
WCHUNK = 40

NBMAX = NQ

_mesh = plsc.VectorSubcoreMesh(core_axis_name="c", subcore_axis_name="s")
_cparams = pltpu.CompilerParams(use_tc_tiling_on_sc=False,
                                needs_layout_passes=False)


@functools.partial(
    pl.kernel,
    mesh=_mesh,
    out_type=[
        jax.ShapeDtypeStruct((2 * 16 * NBMAX, 3, EC), jnp.int32),
        jax.ShapeDtypeStruct((2, 16, 16), jnp.int32),
    ],
    scratch_types=[
        [pltpu.VMEM((3, EC), jnp.int32)] * 2,
        pltpu.VMEM((3, 2 * EC), jnp.int32),
        pltpu.VMEM((16,), jnp.int32),
        [pltpu.SemaphoreType.DMA] * 2,
    ],
    compiler_params=_cparams,
)
def _partition(pk_hbm, parts_hbm, cnt_hbm, idxb, stage, cntv, sem_i):
    cid = lax.axis_index("c")
    sid = lax.axis_index("s")
    base_node = cid * HALF
    region = (cid * 16 + sid) * NBMAX

    for b in range(2):
        pltpu.async_copy(pk_hbm.at[sid + b * 16], idxb[b], sem_i[b])

    def super_body(g, carry):
        n, blk = carry
        for b in range(2):
            q = 2 * g + b
            j = sid + q * 16
            pltpu.make_async_copy(pk_hbm.at[j], idxb[b], sem_i[b]).wait()
            for k in range(EC // 16):
                s16 = pl.ds(k * 16, 16)
                srcv = idxb[b][0, s16]
                dv = idxb[b][1, s16] - base_node
                wv = idxb[b][2, s16]
                ok = (dv >= 0) & (dv < HALF)
                plsc.store_compressed(stage.at[0, pl.ds(n, 16)], srcv,
                                      mask=ok)
                plsc.store_compressed(stage.at[1, pl.ds(n, 16)], dv, mask=ok)
                plsc.store_compressed(stage.at[2, pl.ds(n, 16)], wv, mask=ok)
                n = n + plsc.all_reduce_population_count(ok)[0]
            @pl.when(q + 2 < NQ)
            def _():
                pltpu.async_copy(pk_hbm.at[j + 32], idxb[b], sem_i[b])
            full = n >= EC
            @pl.when(full)
            def _():
                jj = region + blk
                for r in range(3):
                    pltpu.sync_copy(stage.at[r, pl.ds(0, EC)],
                                    parts_hbm.at[jj, r])
                for r in range(3):
                    for k in range(EC // 16):
                        stage[r, pl.ds(k * 16, 16)] = (
                            stage[r, pl.ds(EC + k * 16, 16)])
            n = jnp.where(full, n - EC, n)
            blk = blk + jnp.where(full, 1, 0)
        return n, blk

    n, blk = lax.fori_loop(0, NQ // 2, super_body,
                           (jnp.int32(0), jnp.int32(0)))

    zero16 = jnp.zeros((16,), jnp.int32)
    pad16 = jnp.full((16,), PAD_ROW, jnp.int32)
    for k in range(EC // 16):
        stage[0, pl.ds(n + k * 16, 16)] = zero16
        stage[1, pl.ds(n + k * 16, 16)] = pad16
        stage[2, pl.ds(n + k * 16, 16)] = zero16
    @pl.when(n > 0)
    def _():
        jj = region + blk
        for r in range(3):
            pltpu.sync_copy(stage.at[r, pl.ds(0, EC)], parts_hbm.at[jj, r])
    nb = blk + jnp.where(n > 0, 1, 0)
    cntv[pl.ds(0, 16)] = jnp.full((16,), nb, jnp.int32)
    pltpu.sync_copy(cntv, cnt_hbm.at[cid, sid])


@functools.partial(
    pl.kernel,
    mesh=_mesh,
    out_type=[
        jax.ShapeDtypeStruct((N_NODES, D), jnp.float32),
        jax.ShapeDtypeStruct((N_NODES, D), jnp.float32),
    ],
    scratch_types=[
        pltpu.VMEM_SHARED((ACC_ROWS, D), jnp.float32),
        [pltpu.VMEM((3, EC), jnp.int32)] * 2,
        [pltpu.VMEM((EC,), jnp.float32)] * 2,
        [pltpu.VMEM((EC,), jnp.int32)] * 2,
        [pltpu.VMEM((EC, D), jnp.float32)] * 2,
        pltpu.VMEM((WCHUNK, D), jnp.float32),
        pltpu.VMEM((WCHUNK, D), jnp.float32),
        pltpu.VMEM((16,), jnp.int32),
        [pltpu.SemaphoreType.DMA] * 2,
        [pltpu.SemaphoreType.DMA] * 2,
        [pltpu.SemaphoreType.DMA] * 2,
    ],
    compiler_params=_cparams,
)
def _layer(x_hbm, parts_hbm, cnt_hbm, sum_hbm,
           xo_hbm, so_hbm,
           acc, idxb, weff, ldst, rows, rb, sb, cnts, sem_i, sem_g, sem_s):
    cid = lax.axis_index("c")
    sid = lax.axis_index("s")
    base_node = cid * HALF
    region = (cid * 16 + sid) * NBMAX
    pltpu.sync_copy(cnt_hbm.at[cid, sid], cnts)
    nb = cnts[pl.ds(0, 16)][0]

    def zset(i, _):
        for dd in range(4):
            rb[i, pl.ds(dd * 16, 16)] = jnp.zeros((16,), jnp.float32)
        return 0
    lax.fori_loop(0, WCHUNK, zset, 0)

    def zb(t, _):
        c = sid + t * 16
        pltpu.sync_copy(rb, acc.at[pl.ds(c * WCHUNK, WCHUNK)])
        return 0
    n_z = 39 + jnp.where(sid < 1, 1, 0)
    lax.fori_loop(0, n_z, zb, 0)
    @pl.when(sid == 0)
    def _():
        pltpu.sync_copy(rb.at[pl.ds(0, 8)], acc.at[pl.ds(HALF, 8)])

    plsc.subcore_barrier()

    for b in range(2):
        @pl.when(nb > b)
        def _():
            pltpu.async_copy(parts_hbm.at[region + b], idxb[b], sem_i[b])

    def super_body(g, _):
        for b in range(2):
            q = 2 * g + b
            @pl.when(q < nb)
            def _():
                pltpu.make_async_copy(parts_hbm.at[region + q], idxb[b],
                                      sem_i[b]).wait()
                @pl.when(g > 0)
                def _():
                    pltpu.make_async_copy(rows[b], acc.at[ldst[b]],
                                          sem_s[b]).wait()
                pltpu.async_copy(x_hbm.at[idxb[b].at[0]], rows[b], sem_g[b])

                @plsc.parallel_loop(0, EC // 16, unroll=4)
                def _(k):
                    s16 = pl.ds(k * 16, 16)
                    ldst[b][s16] = idxb[b][1, s16]
                    weff[b][s16] = plsc.bitcast(idxb[b][2, s16], jnp.float32)

        for b in range(2):
            q = 2 * g + b
            @pl.when(q < nb)
            def _():
                pltpu.make_async_copy(x_hbm.at[idxb[b].at[0]], rows[b],
                                      sem_g[b]).wait()
                @pl.when(q + 2 < nb)
                def _():
                    pltpu.async_copy(parts_hbm.at[region + q + 2], idxb[b],
                                     sem_i[b])

                @plsc.parallel_loop(0, EC // 16, unroll=4)
                def _(k):
                    wv = weff[b][pl.ds(k * 16, 16)]
                    for l in range(16):
                        i = k * 16 + l
                        ws = wv[l]
                        for dd in range(4):
                            s16 = pl.ds(dd * 16, 16)
                            rows[b][i, s16] = rows[b][i, s16] * ws

                pltpu.async_copy(rows[b], acc.at[ldst[b]], sem_s[b],
                                 add=True)
        return 0

    lax.fori_loop(0, (nb + 1) // 2, super_body, 0)
    for b in range(2):
        @pl.when(nb > b)
        def _():
            pltpu.make_async_copy(rows[b], acc.at[ldst[b]], sem_s[b]).wait()

    plsc.subcore_barrier()

    def wb(t, _):
        c = sid + t * 16
        r0 = c * WCHUNK
        pltpu.sync_copy(acc.at[pl.ds(r0, WCHUNK)], rb)
        pltpu.sync_copy(sum_hbm.at[pl.ds(base_node + r0, WCHUNK)], sb)

        @plsc.parallel_loop(0, WCHUNK, unroll=4)
        def _(i):
            for dd in range(4):
                s = pl.ds(dd * 16, 16)
                sb[i, s] = sb[i, s] + rb[i, s]

        pltpu.sync_copy(rb, xo_hbm.at[pl.ds(base_node + r0, WCHUNK)])
        pltpu.sync_copy(sb, so_hbm.at[pl.ds(base_node + r0, WCHUNK)])
        return 0
    n_w = 39 + jnp.where(sid < 1, 1, 0)
    lax.fori_loop(0, n_w, wb, 0)


UPT = 32
IPAD = 112


@functools.partial(
    pl.kernel,
    mesh=_mesh,
    out_type=jax.ShapeDtypeStruct((1024, IPAD), jnp.float32),
    scratch_types=[
        pltpu.VMEM((UPT,), jnp.int32),
        pltpu.VMEM((UPT, D), jnp.float32),
        pltpu.VMEM((224,), jnp.int32),
        pltpu.VMEM((2, IPAD), jnp.int32),
        pltpu.VMEM((IPAD, D), jnp.float32),
        pltpu.VMEM((IPAD,), jnp.float32),
        pltpu.SemaphoreType.DMA,
    ],
    compiler_params=_cparams,
)
def _rate(sum_hbm, users_hbm, itemsf_hbm,
          out_hbm,
          uidx, urows, pairb, iidx, irows, rbuf, sem):
    cid = lax.axis_index("c")
    sid = lax.axis_index("s")
    wid = sid * 2 + cid
    ubase = wid * UPT

    pltpu.sync_copy(users_hbm.at[pl.ds(ubase, UPT)], uidx)
    pltpu.async_copy(sum_hbm.at[uidx], urows, sem).wait()

    lane = lax.iota(jnp.int32, 16)

    def pair_body(p, _):
        u_glob = ubase + 2 * p
        pltpu.sync_copy(itemsf_hbm.at[pl.ds(u_glob * 100, 200)],
                        pairb.at[pl.ds(0, 200)])

        for u in range(2):
            def fill(k, _):
                v = pairb[pl.ds(u * 100 + k * 16, 16)]
                v = jnp.clip(v + N_USERS, 0, N_NODES - 1)
                iidx[u, pl.ds(k * 16, 16)] = v
                return 0
            lax.fori_loop(0, IPAD // 16, fill, 0)

        for u in range(2):
            pltpu.async_copy(sum_hbm.at[iidx.at[u]], irows, sem).wait()
            ur = 2 * p + u

            def dot_d(dg, accs):
                uv = urows[ur, pl.ds(dg * 16, 16)]
                for dl in range(16):
                    d = dg * 16 + dl
                    ud = uv[dl]
                    new = []
                    for g in range(IPAD // 16):
                        col = plsc.load_gather(
                            irows,
                            [g * 16 + lane, jnp.full((16,), d, jnp.int32)])
                        new.append(accs[g] + ud * col)
                    accs = tuple(new)
                return accs

            accs = lax.fori_loop(
                0, D // 16, dot_d,
                tuple(jnp.zeros((16,), jnp.float32) for _ in range(IPAD // 16)))

            for g in range(IPAD // 16):
                r = accs[g] * (1.0 / 16.0)
                s = 1.0 / (1.0 + jnp.exp(-r))
                rbuf[pl.ds(g * 16, 16)] = s
            pltpu.sync_copy(rbuf, out_hbm.at[u_glob + u])
        return 0

    lax.fori_loop(0, UPT // 2, pair_body, 0)


def kernel(users, items, edge_index, edge_weight, user_emb, item_emb):
    all_emb = jnp.concatenate([user_emb, item_emb], axis=0)
    src = edge_index[0]
    dst = edge_index[1]
    wbits = lax.bitcast_convert_type(edge_weight, jnp.int32)
    pad = E_PAD - N_EDGES
    zpad = jnp.zeros((pad,), jnp.int32)
    packed = jnp.stack([
        jnp.concatenate([src, zpad]),
        jnp.concatenate([dst, zpad]),
        jnp.concatenate([wbits, zpad]),
    ], axis=0).reshape(3, N_CHUNKS, EC).transpose(1, 0, 2)

    parts, cnts = _partition(packed)
    x = all_emb
    s = all_emb
    for _ in range(3):
        x, s = _layer(x, parts, cnts, s)
    out = _rate(s, users, items.reshape(-1))
    return out[:, :100]


# speedup vs baseline: 8.2167x; 1.0321x over previous
import functools

import jax
import jax.numpy as jnp
from jax import lax
from jax.experimental import pallas as pl
from jax.experimental.pallas import tpu as pltpu
from jax.experimental.pallas import tpu_sc as plsc

N_USERS = 10000
N_ITEMS = 40000
N_NODES = N_USERS + N_ITEMS
D = 64
N_EDGES = 800000
HALF = N_NODES // 2
ACC_ROWS = HALF + 8
PAD_ROW = HALF
EC = 128
NQ = 392
N_CHUNKS = NQ * 16
E_PAD = N_CHUNKS * EC
WCHUNK = 40

NBMAX = NQ

_mesh = plsc.VectorSubcoreMesh(core_axis_name="c", subcore_axis_name="s")
_cparams = pltpu.CompilerParams(use_tc_tiling_on_sc=False,
                                needs_layout_passes=False)


@functools.partial(
    pl.kernel,
    mesh=_mesh,
    out_type=[
        jax.ShapeDtypeStruct((2 * 16 * NBMAX, 3, EC), jnp.int32),
        jax.ShapeDtypeStruct((2, 16, 16), jnp.int32),
    ],
    scratch_types=[
        [pltpu.VMEM((3, EC), jnp.int32)] * 2,
        pltpu.VMEM((3, 2 * EC), jnp.int32),
        pltpu.VMEM((16,), jnp.int32),
        [pltpu.SemaphoreType.DMA] * 2,
    ],
    compiler_params=_cparams,
)
def _partition(pk_hbm, parts_hbm, cnt_hbm, idxb, stage, cntv, sem_i):
    cid = lax.axis_index("c")
    sid = lax.axis_index("s")
    base_node = cid * HALF
    region = (cid * 16 + sid) * NBMAX

    for b in range(2):
        pltpu.async_copy(pk_hbm.at[sid + b * 16], idxb[b], sem_i[b])

    def super_body(g, carry):
        n, blk = carry
        for b in range(2):
            q = 2 * g + b
            j = sid + q * 16
            pltpu.make_async_copy(pk_hbm.at[j], idxb[b], sem_i[b]).wait()
            for k in range(EC // 16):
                s16 = pl.ds(k * 16, 16)
                srcv = idxb[b][0, s16]
                dv = idxb[b][1, s16] - base_node
                wv = idxb[b][2, s16]
                ok = (dv >= 0) & (dv < HALF)
                plsc.store_compressed(stage.at[0, pl.ds(n, 16)], srcv,
                                      mask=ok)
                plsc.store_compressed(stage.at[1, pl.ds(n, 16)], dv, mask=ok)
                plsc.store_compressed(stage.at[2, pl.ds(n, 16)], wv, mask=ok)
                n = n + plsc.all_reduce_population_count(ok)[0]
            @pl.when(q + 2 < NQ)
            def _():
                pltpu.async_copy(pk_hbm.at[j + 32], idxb[b], sem_i[b])
            full = n >= EC
            @pl.when(full)
            def _():
                jj = region + blk
                for r in range(3):
                    pltpu.sync_copy(stage.at[r, pl.ds(0, EC)],
                                    parts_hbm.at[jj, r])
                for r in range(3):
                    for k in range(EC // 16):
                        stage[r, pl.ds(k * 16, 16)] = (
                            stage[r, pl.ds(EC + k * 16, 16)])
            n = jnp.where(full, n - EC, n)
            blk = blk + jnp.where(full, 1, 0)
        return n, blk

    n, blk = lax.fori_loop(0, NQ // 2, super_body,
                           (jnp.int32(0), jnp.int32(0)))

    zero16 = jnp.zeros((16,), jnp.int32)
    pad16 = jnp.full((16,), PAD_ROW, jnp.int32)
    for k in range(EC // 16):
        stage[0, pl.ds(n + k * 16, 16)] = zero16
        stage[1, pl.ds(n + k * 16, 16)] = pad16
        stage[2, pl.ds(n + k * 16, 16)] = zero16
    @pl.when(n > 0)
    def _():
        jj = region + blk
        for r in range(3):
            pltpu.sync_copy(stage.at[r, pl.ds(0, EC)], parts_hbm.at[jj, r])
    nb = blk + jnp.where(n > 0, 1, 0)
    cntv[pl.ds(0, 16)] = jnp.full((16,), nb, jnp.int32)
    pltpu.sync_copy(cntv, cnt_hbm.at[cid, sid])


@functools.partial(
    pl.kernel,
    mesh=_mesh,
    out_type=[
        jax.ShapeDtypeStruct((N_NODES, D), jnp.float32),
        jax.ShapeDtypeStruct((N_NODES, D), jnp.float32),
    ],
    scratch_types=[
        pltpu.VMEM_SHARED((ACC_ROWS, D), jnp.float32),
        [pltpu.VMEM((3, EC), jnp.int32)] * 2,
        [pltpu.VMEM((EC,), jnp.float32)] * 2,
        [pltpu.VMEM((EC,), jnp.int32)] * 2,
        [pltpu.VMEM((EC, D), jnp.float32)] * 2,
        pltpu.VMEM((WCHUNK, D), jnp.float32),
        pltpu.VMEM((WCHUNK, D), jnp.float32),
        pltpu.VMEM((16,), jnp.int32),
        [pltpu.SemaphoreType.DMA] * 2,
        [pltpu.SemaphoreType.DMA] * 2,
        [pltpu.SemaphoreType.DMA] * 2,
    ],
    compiler_params=_cparams,
)
def _layer(x_hbm, parts_hbm, cnt_hbm, sum_hbm,
           xo_hbm, so_hbm,
           acc, idxb, weff, ldst, rows, rb, sb, cnts, sem_i, sem_g, sem_s):
    cid = lax.axis_index("c")
    sid = lax.axis_index("s")
    base_node = cid * HALF
    region = (cid * 16 + sid) * NBMAX
    pltpu.sync_copy(cnt_hbm.at[cid, sid], cnts)
    nb = cnts[pl.ds(0, 16)][0]

    def zset(i, _):
        for dd in range(4):
            rb[i, pl.ds(dd * 16, 16)] = jnp.zeros((16,), jnp.float32)
        return 0
    lax.fori_loop(0, WCHUNK, zset, 0)

    def zb(t, _):
        c = sid + t * 16
        pltpu.sync_copy(rb, acc.at[pl.ds(c * WCHUNK, WCHUNK)])
        return 0
    n_z = 39 + jnp.where(sid < 1, 1, 0)
    lax.fori_loop(0, n_z, zb, 0)
    @pl.when(sid == 0)
    def _():
        pltpu.sync_copy(rb.at[pl.ds(0, 8)], acc.at[pl.ds(HALF, 8)])

    plsc.subcore_barrier()

    for b in range(2):
        @pl.when(nb > b)
        def _():
            pltpu.async_copy(parts_hbm.at[region + b], idxb[b], sem_i[b])

    def super_body(g, _):
        for b in range(2):
            q = 2 * g + b
            @pl.when(q < nb)
            def _():
                pltpu.make_async_copy(parts_hbm.at[region + q], idxb[b],
                                      sem_i[b]).wait()
                @pl.when(g > 0)
                def _():
                    pltpu.make_async_copy(rows[b], acc.at[ldst[b]],
                                          sem_s[b]).wait()
                pltpu.async_copy(x_hbm.at[idxb[b].at[0]], rows[b], sem_g[b])

                @plsc.parallel_loop(0, EC // 16, unroll=4)
                def _(k):
                    s16 = pl.ds(k * 16, 16)
                    ldst[b][s16] = idxb[b][1, s16]
                    weff[b][s16] = plsc.bitcast(idxb[b][2, s16], jnp.float32)

        for b in range(2):
            q = 2 * g + b
            @pl.when(q < nb)
            def _():
                pltpu.make_async_copy(x_hbm.at[idxb[b].at[0]], rows[b],
                                      sem_g[b]).wait()
                @pl.when(q + 2 < nb)
                def _():
                    pltpu.async_copy(parts_hbm.at[region + q + 2], idxb[b],
                                     sem_i[b])

                @plsc.parallel_loop(0, EC // 16, unroll=4)
                def _(k):
                    wv = weff[b][pl.ds(k * 16, 16)]
                    for l in range(16):
                        i = k * 16 + l
                        ws = wv[l]
                        for dd in range(4):
                            s16 = pl.ds(dd * 16, 16)
                            rows[b][i, s16] = rows[b][i, s16] * ws

                pltpu.async_copy(rows[b], acc.at[ldst[b]], sem_s[b],
                                 add=True)
        return 0

    lax.fori_loop(0, (nb + 1) // 2, super_body, 0)
    for b in range(2):
        @pl.when(nb > b)
        def _():
            pltpu.make_async_copy(rows[b], acc.at[ldst[b]], sem_s[b]).wait()

    plsc.subcore_barrier()

    def wb(t, _):
        c = sid + t * 16
        r0 = c * WCHUNK
        pltpu.sync_copy(acc.at[pl.ds(r0, WCHUNK)], rb)
        pltpu.sync_copy(sum_hbm.at[pl.ds(base_node + r0, WCHUNK)], sb)

        @plsc.parallel_loop(0, WCHUNK, unroll=4)
        def _(i):
            for dd in range(4):
                s = pl.ds(dd * 16, 16)
                sb[i, s] = sb[i, s] + rb[i, s]

        pltpu.sync_copy(rb, xo_hbm.at[pl.ds(base_node + r0, WCHUNK)])
        pltpu.sync_copy(sb, so_hbm.at[pl.ds(base_node + r0, WCHUNK)])
        return 0
    n_w = 39 + jnp.where(sid < 1, 1, 0)
    lax.fori_loop(0, n_w, wb, 0)


UPT = 32
IPAD = 112


@functools.partial(
    pl.kernel,
    mesh=_mesh,
    out_type=jax.ShapeDtypeStruct((1024, IPAD), jnp.float32),
    scratch_types=[
        pltpu.VMEM((UPT,), jnp.int32),
        pltpu.VMEM((UPT, D), jnp.float32),
        pltpu.VMEM((UPT * 100 + 16,), jnp.int32),
        pltpu.VMEM((UPT, IPAD), jnp.int32),
        [pltpu.VMEM((IPAD, D), jnp.float32)] * 2,
        [pltpu.VMEM((IPAD,), jnp.float32)] * 2,
        pltpu.SemaphoreType.DMA,
        [pltpu.SemaphoreType.DMA] * 2,
        [pltpu.SemaphoreType.DMA] * 2,
    ],
    compiler_params=_cparams,
)
def _rate(sum_hbm, users_hbm, itemsf_hbm,
          out_hbm,
          uidx, urows, allid, iidx, irows, rbuf, sem, sem_g, sem_o):
    cid = lax.axis_index("c")
    sid = lax.axis_index("s")
    wid = sid * 2 + cid
    ubase = wid * UPT

    pltpu.sync_copy(users_hbm.at[pl.ds(ubase, UPT)], uidx)
    ud_cp = pltpu.async_copy(sum_hbm.at[uidx], urows, sem)
    pltpu.sync_copy(itemsf_hbm.at[pl.ds(ubase * 100, UPT * 100)],
                    allid.at[pl.ds(0, UPT * 100)])

    @plsc.parallel_loop(0, UPT, unroll=2)
    def _(u):
        for k in range(IPAD // 16):
            v = allid[pl.ds(u * 100 + k * 16, 16)]
            v = jnp.clip(v + N_USERS, 0, N_NODES - 1)
            iidx[u, pl.ds(k * 16, 16)] = v

    ud_cp.wait()
    lane = lax.iota(jnp.int32, 16)

    for b in range(2):
        pltpu.async_copy(sum_hbm.at[iidx.at[b]], irows[b], sem_g[b])

    def pair_body(p, _):
        for b in range(2):
            u = 2 * p + b
            pltpu.make_async_copy(sum_hbm.at[iidx.at[u]], irows[b],
                                  sem_g[b]).wait()

            def dot_d(dg, accs):
                uv = urows[u, pl.ds(dg * 16, 16)]
                for dl in range(16):
                    d = dg * 16 + dl
                    ud = uv[dl]
                    new = []
                    for g in range(IPAD // 16):
                        col = plsc.load_gather(
                            irows[b],
                            [g * 16 + lane, jnp.full((16,), d, jnp.int32)])
                        new.append(accs[g] + ud * col)
                    accs = tuple(new)
                return accs

            accs = lax.fori_loop(
                0, D // 16, dot_d,
                tuple(jnp.zeros((16,), jnp.float32) for _ in range(IPAD // 16)))

            @pl.when(p < UPT // 2 - 1)
            def _():
                pltpu.async_copy(sum_hbm.at[iidx.at[u + 2]], irows[b],
                                 sem_g[b])

            @pl.when(p > 0)
            def _():
                pltpu.make_async_copy(rbuf[b], out_hbm.at[ubase + u],
                                      sem_o[b]).wait()
            for g in range(IPAD // 16):
                r = accs[g] * (1.0 / 16.0)
                s = 1.0 / (1.0 + jnp.exp(-r))
                rbuf[b][pl.ds(g * 16, 16)] = s
            pltpu.async_copy(rbuf[b], out_hbm.at[ubase + u], sem_o[b])
        return 0

    lax.fori_loop(0, UPT // 2, pair_body, 0)
    for b in range(2):
        pltpu.make_async_copy(rbuf[b], out_hbm.at[ubase + b],
                              sem_o[b]).wait()


def kernel(users, items, edge_index, edge_weight, user_emb, item_emb):
    all_emb = jnp.concatenate([user_emb, item_emb], axis=0)
    src = edge_index[0]
    dst = edge_index[1]
    wbits = lax.bitcast_convert_type(edge_weight, jnp.int32)
    pad = E_PAD - N_EDGES
    zpad = jnp.zeros((pad,), jnp.int32)
    packed = jnp.stack([
        jnp.concatenate([src, zpad]),
        jnp.concatenate([dst, zpad]),
        jnp.concatenate([wbits, zpad]),
    ], axis=0).reshape(3, N_CHUNKS, EC).transpose(1, 0, 2)

    parts, cnts = _partition(packed)
    x = all_emb
    s = all_emb
    for _ in range(3):
        x, s = _layer(x, parts, cnts, s)
    out = _rate(s, users, items.reshape(-1))
    return out[:, :100]


# speedup vs baseline: 8.3639x; 1.0179x over previous
import functools

import jax
import jax.numpy as jnp
from jax import lax
from jax.experimental import pallas as pl
from jax.experimental.pallas import tpu as pltpu
from jax.experimental.pallas import tpu_sc as plsc

N_USERS = 10000
N_ITEMS = 40000
N_NODES = N_USERS + N_ITEMS
D = 64
N_EDGES = 800000
HALF = N_NODES // 2
ACC_ROWS = HALF + 8
PAD_ROW = HALF
EC = 128
NQ = 392
N_CHUNKS = NQ * 16
E_PAD = N_CHUNKS * EC
WCHUNK = 40

NBMAX = NQ

_mesh = plsc.VectorSubcoreMesh(core_axis_name="c", subcore_axis_name="s")
_cparams = pltpu.CompilerParams(use_tc_tiling_on_sc=False,
                                needs_layout_passes=False)


@functools.partial(
    pl.kernel,
    mesh=_mesh,
    out_type=[
        jax.ShapeDtypeStruct((2 * 16 * NBMAX, 3, EC), jnp.int32),
        jax.ShapeDtypeStruct((2, 16, 16), jnp.int32),
    ],
    scratch_types=[
        [pltpu.VMEM((3, EC), jnp.int32)] * 2,
        pltpu.VMEM((3, 2 * EC), jnp.int32),
        pltpu.VMEM((16,), jnp.int32),
        [pltpu.SemaphoreType.DMA] * 2,
    ],
    compiler_params=_cparams,
)
def _partition(pk_hbm, parts_hbm, cnt_hbm, idxb, stage, cntv, sem_i):
    cid = lax.axis_index("c")
    sid = lax.axis_index("s")
    base_node = cid * HALF
    region = (cid * 16 + sid) * NBMAX

    for b in range(2):
        pltpu.async_copy(pk_hbm.at[sid + b * 16], idxb[b], sem_i[b])

    def super_body(g, carry):
        n, blk = carry
        for b in range(2):
            q = 2 * g + b
            j = sid + q * 16
            pltpu.make_async_copy(pk_hbm.at[j], idxb[b], sem_i[b]).wait()
            for k in range(EC // 16):
                s16 = pl.ds(k * 16, 16)
                srcv = idxb[b][0, s16]
                dv = idxb[b][1, s16] - base_node
                wv = idxb[b][2, s16]
                ok = (dv >= 0) & (dv < HALF)
                plsc.store_compressed(stage.at[0, pl.ds(n, 16)], srcv,
                                      mask=ok)
                plsc.store_compressed(stage.at[1, pl.ds(n, 16)], dv, mask=ok)
                plsc.store_compressed(stage.at[2, pl.ds(n, 16)], wv, mask=ok)
                n = n + plsc.all_reduce_population_count(ok)[0]
            @pl.when(q + 2 < NQ)
            def _():
                pltpu.async_copy(pk_hbm.at[j + 32], idxb[b], sem_i[b])
            full = n >= EC
            @pl.when(full)
            def _():
                jj = region + blk
                for r in range(3):
                    pltpu.sync_copy(stage.at[r, pl.ds(0, EC)],
                                    parts_hbm.at[jj, r])
                for r in range(3):
                    for k in range(EC // 16):
                        stage[r, pl.ds(k * 16, 16)] = (
                            stage[r, pl.ds(EC + k * 16, 16)])
            n = jnp.where(full, n - EC, n)
            blk = blk + jnp.where(full, 1, 0)
        return n, blk

    n, blk = lax.fori_loop(0, NQ // 2, super_body,
                           (jnp.int32(0), jnp.int32(0)))

    zero16 = jnp.zeros((16,), jnp.int32)
    pad16 = jnp.full((16,), PAD_ROW, jnp.int32)
    for k in range(EC // 16):
        stage[0, pl.ds(n + k * 16, 16)] = zero16
        stage[1, pl.ds(n + k * 16, 16)] = pad16
        stage[2, pl.ds(n + k * 16, 16)] = zero16
    @pl.when(n > 0)
    def _():
        jj = region + blk
        for r in range(3):
            pltpu.sync_copy(stage.at[r, pl.ds(0, EC)], parts_hbm.at[jj, r])
    nb = blk + jnp.where(n > 0, 1, 0)
    cntv[pl.ds(0, 16)] = jnp.full((16,), nb, jnp.int32)
    pltpu.sync_copy(cntv, cnt_hbm.at[cid, sid])


@functools.partial(
    pl.kernel,
    mesh=_mesh,
    out_type=[
        jax.ShapeDtypeStruct((N_NODES, D), jnp.float32),
        jax.ShapeDtypeStruct((N_NODES, D), jnp.float32),
    ],
    scratch_types=[
        pltpu.VMEM_SHARED((ACC_ROWS, D), jnp.float32),
        [pltpu.VMEM((3, EC), jnp.int32)] * 2,
        [pltpu.VMEM((EC,), jnp.float32)] * 2,
        [pltpu.VMEM((EC,), jnp.int32)] * 2,
        [pltpu.VMEM((EC, D), jnp.float32)] * 2,
        [pltpu.VMEM((WCHUNK, D), jnp.float32)] * 2,
        [pltpu.VMEM((WCHUNK, D), jnp.float32)] * 2,
        pltpu.VMEM((16,), jnp.int32),
        [pltpu.SemaphoreType.DMA] * 2,
        [pltpu.SemaphoreType.DMA] * 2,
        [pltpu.SemaphoreType.DMA] * 2,
        [pltpu.SemaphoreType.DMA] * 2,
        [pltpu.SemaphoreType.DMA] * 2,
    ],
    compiler_params=_cparams,
)
def _layer(x_hbm, parts_hbm, cnt_hbm, sum_hbm,
           xo_hbm, so_hbm,
           acc, idxb, weff, ldst, rows, rb, sb, cnts,
           sem_i, sem_g, sem_s, sem_x, sem_y):
    cid = lax.axis_index("c")
    sid = lax.axis_index("s")
    base_node = cid * HALF
    region = (cid * 16 + sid) * NBMAX
    pltpu.sync_copy(cnt_hbm.at[cid, sid], cnts)
    nb = cnts[pl.ds(0, 16)][0]

    def zset(i, _):
        for dd in range(4):
            rb[0][i, pl.ds(dd * 16, 16)] = jnp.zeros((16,), jnp.float32)
        return 0
    lax.fori_loop(0, WCHUNK, zset, 0)

    def zb(t, _):
        c = sid + t * 16
        pltpu.sync_copy(rb[0], acc.at[pl.ds(c * WCHUNK, WCHUNK)])
        return 0
    n_z = 39 + jnp.where(sid < 1, 1, 0)
    lax.fori_loop(0, n_z, zb, 0)
    @pl.when(sid == 0)
    def _():
        pltpu.sync_copy(rb[0].at[pl.ds(0, 8)], acc.at[pl.ds(HALF, 8)])

    plsc.subcore_barrier()

    for b in range(2):
        @pl.when(nb > b)
        def _():
            pltpu.async_copy(parts_hbm.at[region + b], idxb[b], sem_i[b])

    def super_body(g, _):
        for b in range(2):
            q = 2 * g + b
            @pl.when(q < nb)
            def _():
                pltpu.make_async_copy(parts_hbm.at[region + q], idxb[b],
                                      sem_i[b]).wait()
                @pl.when(g > 0)
                def _():
                    pltpu.make_async_copy(rows[b], acc.at[ldst[b]],
                                          sem_s[b]).wait()
                pltpu.async_copy(x_hbm.at[idxb[b].at[0]], rows[b], sem_g[b])

                @plsc.parallel_loop(0, EC // 16, unroll=4)
                def _(k):
                    s16 = pl.ds(k * 16, 16)
                    ldst[b][s16] = idxb[b][1, s16]
                    weff[b][s16] = plsc.bitcast(idxb[b][2, s16], jnp.float32)

        for b in range(2):
            q = 2 * g + b
            @pl.when(q < nb)
            def _():
                pltpu.make_async_copy(x_hbm.at[idxb[b].at[0]], rows[b],
                                      sem_g[b]).wait()
                @pl.when(q + 2 < nb)
                def _():
                    pltpu.async_copy(parts_hbm.at[region + q + 2], idxb[b],
                                     sem_i[b])

                @plsc.parallel_loop(0, EC // 16, unroll=4)
                def _(k):
                    wv = weff[b][pl.ds(k * 16, 16)]
                    for l in range(16):
                        i = k * 16 + l
                        ws = wv[l]
                        for dd in range(4):
                            s16 = pl.ds(dd * 16, 16)
                            rows[b][i, s16] = rows[b][i, s16] * ws

                pltpu.async_copy(rows[b], acc.at[ldst[b]], sem_s[b],
                                 add=True)
        return 0

    lax.fori_loop(0, (nb + 1) // 2, super_body, 0)
    for b in range(2):
        @pl.when(nb > b)
        def _():
            pltpu.make_async_copy(rows[b], acc.at[ldst[b]], sem_s[b]).wait()

    plsc.subcore_barrier()

    NW = HALF // WCHUNK

    def wb(g, _):
        for b in range(2):
            t = 2 * g + b
            c = sid + t * 16
            @pl.when(c < NW)
            def _():
                r0 = c * WCHUNK
                @pl.when(g > 0)
                def _():
                    pltpu.make_async_copy(
                        rb[b], xo_hbm.at[pl.ds(base_node + r0, WCHUNK)],
                        sem_x[b]).wait()
                    pltpu.make_async_copy(
                        sb[b], so_hbm.at[pl.ds(base_node + r0, WCHUNK)],
                        sem_y[b]).wait()
                pltpu.sync_copy(acc.at[pl.ds(r0, WCHUNK)], rb[b])
                pltpu.sync_copy(sum_hbm.at[pl.ds(base_node + r0, WCHUNK)],
                                sb[b])

                @plsc.parallel_loop(0, WCHUNK, unroll=4)
                def _(i):
                    for dd in range(4):
                        s = pl.ds(dd * 16, 16)
                        sb[b][i, s] = sb[b][i, s] + rb[b][i, s]

                pltpu.async_copy(
                    rb[b], xo_hbm.at[pl.ds(base_node + r0, WCHUNK)],
                    sem_x[b])
                pltpu.async_copy(
                    sb[b], so_hbm.at[pl.ds(base_node + r0, WCHUNK)],
                    sem_y[b])
        return 0

    lax.fori_loop(0, (NW + 31) // 32, wb, 0)
    for b in range(2):
        pltpu.make_async_copy(rb[b], xo_hbm.at[pl.ds(base_node, WCHUNK)],
                              sem_x[b]).wait()
        pltpu.make_async_copy(sb[b], so_hbm.at[pl.ds(base_node, WCHUNK)],
                              sem_y[b]).wait()


UPT = 32
IPAD = 112


@functools.partial(
    pl.kernel,
    mesh=_mesh,
    out_type=jax.ShapeDtypeStruct((1024, IPAD), jnp.float32),
    scratch_types=[
        pltpu.VMEM((UPT,), jnp.int32),
        pltpu.VMEM((UPT, D), jnp.float32),
        pltpu.VMEM((UPT * 100 + 16,), jnp.int32),
        pltpu.VMEM((UPT, IPAD), jnp.int32),
        [pltpu.VMEM((IPAD, D), jnp.float32)] * 2,
        [pltpu.VMEM((IPAD,), jnp.float32)] * 2,
        pltpu.SemaphoreType.DMA,
        [pltpu.SemaphoreType.DMA] * 2,
        [pltpu.SemaphoreType.DMA] * 2,
    ],
    compiler_params=_cparams,
)
def _rate(sum_hbm, users_hbm, itemsf_hbm,
          out_hbm,
          uidx, urows, allid, iidx, irows, rbuf, sem, sem_g, sem_o):
    cid = lax.axis_index("c")
    sid = lax.axis_index("s")
    wid = sid * 2 + cid
    ubase = wid * UPT

    pltpu.sync_copy(users_hbm.at[pl.ds(ubase, UPT)], uidx)
    ud_cp = pltpu.async_copy(sum_hbm.at[uidx], urows, sem)
    pltpu.sync_copy(itemsf_hbm.at[pl.ds(ubase * 100, UPT * 100)],
                    allid.at[pl.ds(0, UPT * 100)])

    @plsc.parallel_loop(0, UPT, unroll=2)
    def _(u):
        for k in range(IPAD // 16):
            v = allid[pl.ds(u * 100 + k * 16, 16)]
            v = jnp.clip(v + N_USERS, 0, N_NODES - 1)
            iidx[u, pl.ds(k * 16, 16)] = v

    ud_cp.wait()
    lane = lax.iota(jnp.int32, 16)

    for b in range(2):
        pltpu.async_copy(sum_hbm.at[iidx.at[b]], irows[b], sem_g[b])

    def pair_body(p, _):
        for b in range(2):
            u = 2 * p + b
            pltpu.make_async_copy(sum_hbm.at[iidx.at[u]], irows[b],
                                  sem_g[b]).wait()

            def dot_d(dg, accs):
                uv = urows[u, pl.ds(dg * 16, 16)]
                for dl in range(16):
                    d = dg * 16 + dl
                    ud = uv[dl]
                    new = []
                    for g in range(IPAD // 16):
                        col = plsc.load_gather(
                            irows[b],
                            [g * 16 + lane, jnp.full((16,), d, jnp.int32)])
                        new.append(accs[g] + ud * col)
                    accs = tuple(new)
                return accs

            accs = lax.fori_loop(
                0, D // 16, dot_d,
                tuple(jnp.zeros((16,), jnp.float32) for _ in range(IPAD // 16)))

            @pl.when(p < UPT // 2 - 1)
            def _():
                pltpu.async_copy(sum_hbm.at[iidx.at[u + 2]], irows[b],
                                 sem_g[b])

            @pl.when(p > 0)
            def _():
                pltpu.make_async_copy(rbuf[b], out_hbm.at[ubase + u],
                                      sem_o[b]).wait()
            for g in range(IPAD // 16):
                r = accs[g] * (1.0 / 16.0)
                s = 1.0 / (1.0 + jnp.exp(-r))
                rbuf[b][pl.ds(g * 16, 16)] = s
            pltpu.async_copy(rbuf[b], out_hbm.at[ubase + u], sem_o[b])
        return 0

    lax.fori_loop(0, UPT // 2, pair_body, 0)
    for b in range(2):
        pltpu.make_async_copy(rbuf[b], out_hbm.at[ubase + b],
                              sem_o[b]).wait()


def kernel(users, items, edge_index, edge_weight, user_emb, item_emb):
    all_emb = jnp.concatenate([user_emb, item_emb], axis=0)
    src = edge_index[0]
    dst = edge_index[1]
    wbits = lax.bitcast_convert_type(edge_weight, jnp.int32)
    pad = E_PAD - N_EDGES
    zpad = jnp.zeros((pad,), jnp.int32)
    packed = jnp.stack([
        jnp.concatenate([src, zpad]),
        jnp.concatenate([dst, zpad]),
        jnp.concatenate([wbits, zpad]),
    ], axis=0).reshape(3, N_CHUNKS, EC).transpose(1, 0, 2)

    parts, cnts = _partition(packed)
    x = all_emb
    s = all_emb
    for _ in range(3):
        x, s = _layer(x, parts, cnts, s)
    out = _rate(s, users, items.reshape(-1))
    return out[:, :100]
